# Initial kernel scaffold; baseline (speedup 1.0000x reference)
import jax
import jax.numpy as jnp
from jax.experimental import pallas as pl


def kernel(x, neighbor_idx, W1, g1, b1, W2, g2, b2, W3, g3, b3, W4, g4, b4, Wh1, gh, bh, Wh2, bh2):
    raise NotImplementedError("write your pallas kernel here")


# speedup vs baseline: 5.7634x; 5.7634x over previous
import functools
import math

import jax
import jax.numpy as jnp
from jax import lax
from jax.experimental import pallas as pl
from jax.experimental.pallas import tpu as pltpu
from jax.experimental.pallas import tpu_sc as plsc

E = 100000
EP = 100352
NW = 32
RA = EP // NW
K1, K1P = 1500, 1536
K2, K2P = 750, 768
K3, K3P = 375, 384
KB = K1P // NW
KEEPN = 2048
BN_INV = 1.0 / math.sqrt(1.0 + 1e-5)
C_BLK = 2048
F32 = jnp.float32
I32 = jnp.int32


def _mesh():
    return plsc.VectorSubcoreMesh(core_axis_name="c", subcore_axis_name="s")


def _wid():
    return lax.axis_index("s") * 2 + lax.axis_index("c")


@functools.cache
def _sc_gather_nbrs_fn():
    @functools.partial(
        pl.kernel, mesh=_mesh(),
        compiler_params=pltpu.CompilerParams(use_tc_tiling_on_sc=False),
        out_type=[jax.ShapeDtypeStruct((EP, 16), F32)] * 4,
        scratch_types=[pltpu.VMEM((RA,), I32), pltpu.VMEM((RA, 16), F32),
                       pltpu.SemaphoreType.DMA],
    )
    def _sc_gather_nbrs(x16, i0, i1, i2, i3, o0, o1, o2, o3,
                        idx_v, rows_v, sem):
        base = _wid() * RA
        for ih, oh in ((i0, o0), (i1, o1), (i2, o2), (i3, o3)):
            pltpu.sync_copy(ih.at[pl.ds(base, RA)], idx_v)
            pltpu.async_copy(x16.at[idx_v], rows_v, sem).wait()
            pltpu.sync_copy(rows_v, oh.at[pl.ds(base, RA)])
    return _sc_gather_nbrs


def _conv1_block(xc, n0, n1, n2, n3, w, g, b):
    mn01 = jnp.minimum(n0[:, 0:5], n1[:, 0:5])
    mx01 = jnp.maximum(n0[:, 0:5], n1[:, 0:5])
    mn23 = jnp.minimum(n2[:, 0:5], n3[:, 0:5])
    mx23 = jnp.maximum(n2[:, 0:5], n3[:, 0:5])
    z = jnp.zeros((xc.shape[0], 7), F32)
    feats = jnp.concatenate([xc[:, 0:5], mn01, mx01, mn23, mx23, z], axis=1)
    h = lax.dot_general(feats, w, (((1,), (0,)), ((), ())),
                        preferred_element_type=F32)
    h = (h * BN_INV) * g[0:1, :] + b[0:1, :]
    return jnp.maximum(h, 0.0)


def _c_body(x_ref, n0_ref, n1_ref, n2_ref, n3_ref, w_ref, g_ref, b_ref,
            out_ref):
    h = _conv1_block(x_ref[...], n0_ref[...], n1_ref[...], n2_ref[...],
                     n3_ref[...], w_ref[...], g_ref, b_ref)
    s = jnp.sqrt(jnp.sum(h * h, axis=1))
    out_ref[...] = s.reshape(C_BLK // 128, 128)


def _scores_call(x16, nbrf, w1p, g1r, b1r):
    blk16 = pl.BlockSpec((C_BLK, 16), lambda i: (i, 0))
    return pl.pallas_call(
        _c_body,
        grid=(EP // C_BLK,),
        in_specs=[
            blk16, blk16, blk16, blk16, blk16,
            pl.BlockSpec((32, 64), lambda i: (0, 0)),
            pl.BlockSpec((8, 64), lambda i: (0, 0)),
            pl.BlockSpec((8, 64), lambda i: (0, 0)),
        ],
        out_specs=pl.BlockSpec((C_BLK // 128, 128), lambda i: (i, 0)),
        out_shape=jax.ShapeDtypeStruct((EP // 128, 128), F32),
    )(x16, *nbrf, w1p, g1r, b1r)


def _excl_cumsum(a):
    r = a.shape[0]
    u = (lax.broadcasted_iota(I32, (128, 128), 0)
         < lax.broadcasted_iota(I32, (128, 128), 1)).astype(F32)
    within = lax.dot_general(a, u, (((1,), (0,)), ((), ())),
                             preferred_element_type=F32,
                             precision=lax.Precision.HIGHEST)
    rs = jnp.sum(a, axis=1, keepdims=True)
    slt = (lax.broadcasted_iota(I32, (r, r), 0)
           > lax.broadcasted_iota(I32, (r, r), 1)).astype(F32)
    rowex = lax.dot_general(slt, rs, (((1,), (0,)), ((), ())),
                            preferred_element_type=F32,
                            precision=lax.Precision.HIGHEST)
    return rowex + within


def _select_topk(scores2d, valid, k):
    ints = jnp.where(valid, lax.bitcast_convert_type(scores2d, I32),
                     jnp.full(scores2d.shape, -1, I32))

    def bs(_, c):
        lo, hi = c
        d = hi - lo
        mid = lo + (d >> 1) + (d & 1)
        cnt = jnp.sum((ints >= mid).astype(I32))
        ok = cnt >= k
        return (jnp.where(ok, mid, lo), jnp.where(ok, hi, mid - 1))

    lo, _ = lax.fori_loop(0, 31, bs, (jnp.int32(0), jnp.int32(2**31 - 1)))
    gt = ints > lo
    eq = ints == lo
    need = (k - jnp.sum(gt.astype(I32))).astype(F32)
    eqr = _excl_cumsum(eq.astype(F32))
    mask = gt | (eq & (eqr < need))
    prefix = _excl_cumsum(mask.astype(F32))
    return mask, prefix


def _d_body(s_ref, out_ref):
    sc = s_ref[...]
    flat = (lax.broadcasted_iota(I32, sc.shape, 0) * 128
            + lax.broadcasted_iota(I32, sc.shape, 1))
    mask, prefix = _select_topk(sc, flat < E, K1)
    out_ref[...] = jnp.where(mask, prefix.astype(I32),
                             jnp.full(sc.shape, -1, I32))


def _remap_call(scores):
    return pl.pallas_call(
        _d_body,
        out_shape=jax.ShapeDtypeStruct((EP // 128, 128), I32),
    )(scores)


@functools.cache
def _sc_scatter_keep_fn():
    @functools.partial(
        pl.kernel, mesh=_mesh(),
        compiler_params=pltpu.CompilerParams(use_tc_tiling_on_sc=False),
        out_type=jax.ShapeDtypeStruct((KEEPN,), I32),
        scratch_types=[pltpu.VMEM((RA,), I32), pltpu.VMEM((RA,), I32),
                       pltpu.VMEM((RA,), I32), pltpu.SemaphoreType.DMA],
    )
    def _sc_scatter_keep(remap, keep, rem_v, dest_v, val_v, sem):
        base = _wid() * RA
        pltpu.sync_copy(remap.at[pl.ds(base, RA)], rem_v)

        def step(s, _):
            r = rem_v[pl.ds(s * 16, 16)]
            gi = base + s * 16 + lax.iota(I32, 16)
            dest_v[pl.ds(s * 16, 16)] = jnp.where(
                r >= 0, r, jnp.full((16,), KEEPN - 1, I32))
            val_v[pl.ds(s * 16, 16)] = gi
            return 0

        lax.fori_loop(0, RA // 16, step, 0)
        pltpu.async_copy(val_v, keep.at[dest_v], sem).wait()
    return _sc_scatter_keep


@functools.cache
def _sc_gather_kept_fn():
  @functools.partial(
    pl.kernel, mesh=_mesh(),
    compiler_params=pltpu.CompilerParams(use_tc_tiling_on_sc=False),
    out_type=[jax.ShapeDtypeStruct((K1P, 16), F32)] * 5
             + [jax.ShapeDtypeStruct((K1P,), I32)] * 4,
    scratch_types=[pltpu.VMEM((KB,), I32), pltpu.VMEM((KB,), I32),
                   pltpu.VMEM((KB, 16), F32),
                   pltpu.VMEM((KB,), I32), pltpu.VMEM((KB,), I32),
                   pltpu.VMEM((KB,), I32), pltpu.SemaphoreType.DMA],
  )
  def _sc_gather_kept(keep, x16, n0, n1, n2, n3, nb0, nb1, nb2, nb3, remap,
                      xk, nk0, nk1, nk2, nk3, nr0, nr1, nr2, nr3,
                      kraw, kidx, rowbuf, cb, cidx, rb, sem):
    base = _wid() * KB
    pltpu.sync_copy(keep.at[pl.ds(base, KB)], kraw)
    for s in range(KB // 16):
        v = kraw[pl.ds(s * 16, 16)]
        kidx[pl.ds(s * 16, 16)] = jnp.minimum(
            jnp.maximum(v, 0), jnp.full((16,), E - 1, I32))
    for tab, dst in ((x16, xk), (n0, nk0), (n1, nk1), (n2, nk2), (n3, nk3)):
        pltpu.async_copy(tab.at[kidx], rowbuf, sem).wait()
        pltpu.sync_copy(rowbuf, dst.at[pl.ds(base, KB)])
    for nb, nr in ((nb0, nr0), (nb1, nr1), (nb2, nr2), (nb3, nr3)):
        pltpu.async_copy(nb.at[kidx], cb, sem).wait()
        for s in range(KB // 16):
            v = cb[pl.ds(s * 16, 16)]
            cidx[pl.ds(s * 16, 16)] = jnp.minimum(
                jnp.maximum(v, 0), jnp.full((16,), E - 1, I32))
        pltpu.async_copy(remap.at[cidx], rb, sem).wait()
        pltpu.sync_copy(rb, nr.at[pl.ds(base, KB)])
  return _sc_gather_kept


def _matmul(a, b):
    return lax.dot_general(a, b, (((1,), (0,)), ((), ())),
                           preferred_element_type=F32,
                           precision=lax.Precision.HIGHEST)


def _gather_rows(table, q):
    n, c = table.shape
    acc = jnp.zeros((q.shape[0], c), F32)
    for ch in range(n // 128):
        ioc = lax.broadcasted_iota(I32, (1, 128), 1) + ch * 128
        oh = (q == ioc).astype(F32)
        acc = acc + _matmul(oh, table[ch * 128:(ch + 1) * 128, :])
    return acc


def _gather_vals(vals2d, q):
    r = vals2d.shape[0]
    acc = jnp.zeros((q.shape[0], 1), F32)
    for ch in range(r):
        ioc = lax.broadcasted_iota(I32, (1, 128), 1) + ch * 128
        oh = (q == ioc).astype(F32)
        acc = acc + lax.dot_general(oh, vals2d[ch:ch + 1, :],
                                    (((1,), (1,)), ((), ())),
                                    preferred_element_type=F32,
                                    precision=lax.Precision.HIGHEST)
    return acc


def _compact_rows(table, prefix, mask, kp):
    n, c = table.shape
    jio = lax.broadcasted_iota(I32, (kp, 1), 0).astype(F32)
    acc = jnp.zeros((kp, c), F32)
    for ch in range(n // 128):
        pr = prefix[ch:ch + 1, :]
        mr = mask[ch:ch + 1, :]
        oh = jnp.where(mr, (jio == pr).astype(F32), jnp.zeros((kp, 128), F32))
        acc = acc + _matmul(oh, table[ch * 128:(ch + 1) * 128, :])
    return acc


def _bn_relu(h, g, b):
    return jnp.maximum((h * BN_INV) * g + b, 0.0)


def _matmul_conv(a, b):
    return lax.dot_general(a, b, (((1,), (0,)), ((), ())),
                           preferred_element_type=F32)


def _conv_tail(h, nr4, w, g, b, valid_n):
    n = h.shape[0]
    iot = lax.broadcasted_iota(I32, (n, 1), 0)
    cols = []
    for j in range(4):
        c = nr4[:, j:j + 1]
        cols.append(jnp.where(c < 0, iot, c))
    q = jnp.concatenate(cols, axis=0)
    gat = _gather_rows(h, q)
    g0, g1_, g2_, g3_ = (gat[j * n:(j + 1) * n, :] for j in range(4))
    feats = jnp.concatenate([
        h, jnp.minimum(g0, g1_), jnp.maximum(g0, g1_),
        jnp.minimum(g2_, g3_), jnp.maximum(g2_, g3_)], axis=1)
    return _bn_relu(_matmul_conv(feats, w), g, b)


def _pool_tail(h, nr4, n_valid, k, kp):
    n, c = h.shape
    r = n // 128
    s = jnp.sqrt(jnp.sum(h * h, axis=1))
    s2d = s.reshape(r, 128)
    flat = (lax.broadcasted_iota(I32, (r, 128), 0) * 128
            + lax.broadcasted_iota(I32, (r, 128), 1))
    valid = flat < n_valid
    s2d = jnp.where(valid, s2d, jnp.zeros((r, 128), F32))
    mask, prefix = _select_topk(s2d, valid, k)
    iot = lax.broadcasted_iota(I32, (n, 1), 0)
    nr4f = jnp.where(nr4 < 0, jnp.broadcast_to(iot, nr4.shape), nr4)
    table = jnp.concatenate([h, nr4f.astype(F32)], axis=1)
    comp = _compact_rows(table, prefix, mask, kp)
    hp = comp[:, :c]
    nbrk = comp[:, c:c + 4].astype(I32)
    remap = jnp.where(mask, prefix, jnp.full((r, 128), -1.0, F32))
    qs = jnp.concatenate([nbrk[:, j:j + 1] for j in range(4)], axis=0)
    qs = jnp.minimum(jnp.maximum(qs, 0), n - 1)
    nrg = _gather_vals(remap, qs)
    nr4p = jnp.concatenate(
        [nrg[j * kp:(j + 1) * kp, :] for j in range(4)], axis=1).astype(I32)
    return hp, nr4p


def _e_body(xk_ref, nk0_ref, nk1_ref, nk2_ref, nk3_ref, nr_ref,
            w1_ref, g1_ref, b1_ref,
            w2_ref, g2_ref, b2_ref, w3_ref, g3_ref, b3_ref,
            w4_ref, g4_ref, b4_ref, wh1_ref, gh_ref, bh_ref,
            wh2_ref, bh2_ref, out_ref):
    nr4 = nr_ref[...]
    h1 = _conv1_block(xk_ref[...], nk0_ref[...], nk1_ref[...], nk2_ref[...],
                      nk3_ref[...], w1_ref[...], g1_ref, b1_ref)
    h2 = _conv_tail(h1, nr4, w2_ref[...], g2_ref[0:1, :], b2_ref[0:1, :], K1)
    h2p, nr4b = _pool_tail(h2, nr4, K1, K2, K2P)
    h3 = _conv_tail(h2p, nr4b, w3_ref[...], g3_ref[0:1, :], b3_ref[0:1, :], K2)
    h3p, nr4c = _pool_tail(h3, nr4b, K2, K3, K3P)
    h4 = _conv_tail(h3p, nr4c, w4_ref[...], g4_ref[0:1, :], b4_ref[0:1, :], K3)
    vio = lax.broadcasted_iota(I32, (K3P, 1), 0) < K3
    hm = jnp.where(vio, h4, jnp.full(h4.shape, -1.0, F32))
    gmax = jnp.max(hm, axis=0, keepdims=True)
    z = _bn_relu(_matmul_conv(gmax, wh1_ref[...]), gh_ref[0:1, :],
                 bh_ref[0:1, :])
    logits = _matmul_conv(z, wh2_ref[...]) + bh2_ref[0:1, :]
    out_ref[...] = jnp.broadcast_to(logits, (8, 128))


def _tail_call(xk, nks, nrm, w1p, g1r, b1r, w2, g2r, b2r, w3, g3r, b3r,
               w4, g4r, b4r, wh1, ghr, bhr, wh2p, bh2r):
    return pl.pallas_call(
        _e_body,
        out_shape=jax.ShapeDtypeStruct((8, 128), F32),
    )(xk, *nks, nrm, w1p, g1r, b1r, w2, g2r, b2r, w3, g3r, b3r,
      w4, g4r, b4r, wh1, ghr, bhr, wh2p, bh2r)


def _row8(v, width):
    return jnp.broadcast_to(v.reshape(1, -1), (8, width))


def kernel(x, neighbor_idx, W1, g1, b1, W2, g2, b2, W3, g3, b3,
           W4, g4, b4, Wh1, gh, bh, Wh2, bh2):
    x16 = jnp.zeros((EP, 16), F32).at[:E, :5].set(x)
    nbc = [jnp.zeros((EP,), I32).at[:E].set(neighbor_idx[:, j])
           for j in range(4)]
    w1p = jnp.zeros((32, 64), F32).at[:25, :].set(W1)
    wh2p = jnp.zeros((256, 128), F32).at[:, :30].set(Wh2)
    g1r, b1r = _row8(g1, 64), _row8(b1, 64)
    g2r, b2r = _row8(g2, 128), _row8(b2, 128)
    g3r, b3r = _row8(g3, 256), _row8(b3, 256)
    g4r, b4r = _row8(g4, 512), _row8(b4, 512)
    ghr, bhr = _row8(gh, 256), _row8(bh, 256)
    bh2r = jnp.zeros((8, 128), F32).at[:, :30].set(
        jnp.broadcast_to(bh2.reshape(1, -1), (8, 30)))

    nbrf = _sc_gather_nbrs_fn()(x16, *nbc)
    scores = _scores_call(x16, nbrf, w1p, g1r, b1r)
    remap2d = _remap_call(scores)
    remap = remap2d.reshape(EP)
    keep = _sc_scatter_keep_fn()(remap)
    xk, nk0, nk1, nk2, nk3, nr0, nr1, nr2, nr3 = _sc_gather_kept_fn()(
        keep, x16, *nbrf, *nbc, remap)
    nrm = jnp.stack([nr0, nr1, nr2, nr3], axis=1)
    out = _tail_call(xk, (nk0, nk1, nk2, nk3), nrm, w1p, g1r, b1r,
                     W2, g2r, b2r,
                     W3, g3r, b3r, W4, g4r, b4r, Wh1, ghr, bhr, wh2p, bh2r)
    return out[:1, :30]


# speedup vs baseline: 108.2391x; 18.7803x over previous
import functools
import math

import jax
import jax.numpy as jnp
from jax import lax
from jax.experimental import pallas as pl
from jax.experimental.pallas import tpu as pltpu
from jax.experimental.pallas import tpu_sc as plsc

E = 100000
EP = 100352
NW = 32
RA = EP // NW
K1, K1P = 1500, 1536
K2, K2P = 750, 768
K3, K3P = 375, 384
KB = K1P // NW
KEEPN = 2048
BN_INV = 1.0 / math.sqrt(1.0 + 1e-5)
C_BLK = 2048
F32 = jnp.float32
I32 = jnp.int32


def _mesh():
    return plsc.VectorSubcoreMesh(core_axis_name="c", subcore_axis_name="s")


def _wid():
    return lax.axis_index("s") * 2 + lax.axis_index("c")


@functools.cache
def _sc_gather_nbrs_fn():
    @functools.partial(
        pl.kernel, mesh=_mesh(),
        compiler_params=pltpu.CompilerParams(use_tc_tiling_on_sc=False),
        out_type=[jax.ShapeDtypeStruct((EP, 16), F32)] * 4,
        scratch_types=[pltpu.VMEM((RA,), I32), pltpu.VMEM((RA, 16), F32),
                       pltpu.SemaphoreType.DMA],
    )
    def _sc_gather_nbrs(x16, i0, i1, i2, i3, o0, o1, o2, o3,
                        idx_v, rows_v, sem):
        base = _wid() * RA
        for ih, oh in ((i0, o0), (i1, o1), (i2, o2), (i3, o3)):
            pltpu.sync_copy(ih.at[pl.ds(base, RA)], idx_v)
            pltpu.async_copy(x16.at[idx_v], rows_v, sem).wait()
            pltpu.sync_copy(rows_v, oh.at[pl.ds(base, RA)])
    return _sc_gather_nbrs


def _conv1_block(xc, n0, n1, n2, n3, w, g, b):
    mn01 = jnp.minimum(n0[:, 0:5], n1[:, 0:5])
    mx01 = jnp.maximum(n0[:, 0:5], n1[:, 0:5])
    mn23 = jnp.minimum(n2[:, 0:5], n3[:, 0:5])
    mx23 = jnp.maximum(n2[:, 0:5], n3[:, 0:5])
    z = jnp.zeros((xc.shape[0], 7), F32)
    feats = jnp.concatenate([xc[:, 0:5], mn01, mx01, mn23, mx23, z], axis=1)
    h = lax.dot_general(feats, w, (((1,), (0,)), ((), ())),
                        preferred_element_type=F32)
    h = (h * BN_INV) * g[0:1, :] + b[0:1, :]
    return jnp.maximum(h, 0.0)


def _c_body(x_ref, n0_ref, n1_ref, n2_ref, n3_ref, w_ref, g_ref, b_ref,
            out_ref):
    h = _conv1_block(x_ref[...], n0_ref[...], n1_ref[...], n2_ref[...],
                     n3_ref[...], w_ref[...], g_ref, b_ref)
    s = jnp.sqrt(jnp.sum(h * h, axis=1))
    out_ref[...] = s.reshape(C_BLK // 128, 128)


def _scores_call(x16, nbrf, w1p, g1r, b1r):
    blk16 = pl.BlockSpec((C_BLK, 16), lambda i: (i, 0))
    return pl.pallas_call(
        _c_body,
        grid=(EP // C_BLK,),
        in_specs=[
            blk16, blk16, blk16, blk16, blk16,
            pl.BlockSpec((32, 64), lambda i: (0, 0)),
            pl.BlockSpec((8, 64), lambda i: (0, 0)),
            pl.BlockSpec((8, 64), lambda i: (0, 0)),
        ],
        out_specs=pl.BlockSpec((C_BLK // 128, 128), lambda i: (i, 0)),
        out_shape=jax.ShapeDtypeStruct((EP // 128, 128), F32),
    )(x16, *nbrf, w1p, g1r, b1r)


def _excl_cumsum(a):
    r = a.shape[0]
    u = (lax.broadcasted_iota(I32, (128, 128), 0)
         < lax.broadcasted_iota(I32, (128, 128), 1)).astype(F32)
    within = lax.dot_general(a, u, (((1,), (0,)), ((), ())),
                             preferred_element_type=F32,
                             precision=lax.Precision.HIGHEST)
    rs = jnp.sum(a, axis=1, keepdims=True)
    slt = (lax.broadcasted_iota(I32, (r, r), 0)
           > lax.broadcasted_iota(I32, (r, r), 1)).astype(F32)
    rowex = lax.dot_general(slt, rs, (((1,), (0,)), ((), ())),
                            preferred_element_type=F32,
                            precision=lax.Precision.HIGHEST)
    return rowex + within


def _select_topk(scores2d, valid, k):
    ints = jnp.where(valid, lax.bitcast_convert_type(scores2d, I32),
                     jnp.full(scores2d.shape, -1, I32))

    def bs(_, c):
        lo, hi = c
        d = hi - lo
        mid = lo + (d >> 1) + (d & 1)
        cnt = jnp.sum((ints >= mid).astype(I32))
        ok = cnt >= k
        return (jnp.where(ok, mid, lo), jnp.where(ok, hi, mid - 1))

    lo, _ = lax.fori_loop(0, 31, bs, (jnp.int32(0), jnp.int32(2**31 - 1)))
    gt = ints > lo
    eq = ints == lo
    need = (k - jnp.sum(gt.astype(I32))).astype(F32)
    eqr = _excl_cumsum(eq.astype(F32))
    mask = gt | (eq & (eqr < need))
    prefix = _excl_cumsum(mask.astype(F32))
    return mask, prefix


def _d_body(s_ref, out_ref):
    sc = s_ref[...]
    flat = (lax.broadcasted_iota(I32, sc.shape, 0) * 128
            + lax.broadcasted_iota(I32, sc.shape, 1))
    mask, prefix = _select_topk(sc, flat < E, K1)
    out_ref[...] = jnp.where(mask, prefix.astype(I32),
                             jnp.full(sc.shape, -1, I32))


def _remap_call(scores):
    return pl.pallas_call(
        _d_body,
        out_shape=jax.ShapeDtypeStruct((EP // 128, 128), I32),
    )(scores)


@functools.cache
def _sc_scatter_keep_fn():
    @functools.partial(
        pl.kernel, mesh=_mesh(),
        compiler_params=pltpu.CompilerParams(use_tc_tiling_on_sc=False),
        out_type=jax.ShapeDtypeStruct((KEEPN + EP,), I32),
        scratch_types=[pltpu.VMEM((RA,), I32), pltpu.VMEM((RA,), I32),
                       pltpu.VMEM((RA,), I32), pltpu.SemaphoreType.DMA],
    )
    def _sc_scatter_keep(remap, keep, rem_v, dest_v, val_v, sem):
        base = _wid() * RA
        pltpu.sync_copy(remap.at[pl.ds(base, RA)], rem_v)

        def step(s, _):
            r = rem_v[pl.ds(s * 16, 16)]
            gi = base + s * 16 + lax.iota(I32, 16)
            dest_v[pl.ds(s * 16, 16)] = jnp.where(r >= 0, r, KEEPN + gi)
            val_v[pl.ds(s * 16, 16)] = gi
            return 0

        lax.fori_loop(0, RA // 16, step, 0)
        pltpu.async_copy(val_v, keep.at[dest_v], sem).wait()
    return _sc_scatter_keep


@functools.cache
def _sc_gather_kept_fn():
  @functools.partial(
    pl.kernel, mesh=_mesh(),
    compiler_params=pltpu.CompilerParams(use_tc_tiling_on_sc=False),
    out_type=[jax.ShapeDtypeStruct((K1P, 16), F32)] * 5
             + [jax.ShapeDtypeStruct((K1P,), I32)] * 4,
    scratch_types=[pltpu.VMEM((KB,), I32), pltpu.VMEM((KB,), I32),
                   pltpu.VMEM((KB, 16), F32),
                   pltpu.VMEM((KB,), I32), pltpu.VMEM((KB,), I32),
                   pltpu.VMEM((KB,), I32), pltpu.SemaphoreType.DMA],
  )
  def _sc_gather_kept(keep, x16, n0, n1, n2, n3, nb0, nb1, nb2, nb3, remap,
                      xk, nk0, nk1, nk2, nk3, nr0, nr1, nr2, nr3,
                      kraw, kidx, rowbuf, cb, cidx, rb, sem):
    base = _wid() * KB
    pltpu.sync_copy(keep.at[pl.ds(base, KB)], kraw)
    for s in range(KB // 16):
        v = kraw[pl.ds(s * 16, 16)]
        kidx[pl.ds(s * 16, 16)] = jnp.minimum(
            jnp.maximum(v, 0), jnp.full((16,), E - 1, I32))
    for tab, dst in ((x16, xk), (n0, nk0), (n1, nk1), (n2, nk2), (n3, nk3)):
        pltpu.async_copy(tab.at[kidx], rowbuf, sem).wait()
        pltpu.sync_copy(rowbuf, dst.at[pl.ds(base, KB)])
    for nb, nr in ((nb0, nr0), (nb1, nr1), (nb2, nr2), (nb3, nr3)):
        pltpu.async_copy(nb.at[kidx], cb, sem).wait()
        for s in range(KB // 16):
            v = cb[pl.ds(s * 16, 16)]
            cidx[pl.ds(s * 16, 16)] = jnp.minimum(
                jnp.maximum(v, 0), jnp.full((16,), E - 1, I32))
        pltpu.async_copy(remap.at[cidx], rb, sem).wait()
        pltpu.sync_copy(rb, nr.at[pl.ds(base, KB)])
  return _sc_gather_kept


def _matmul(a, b):
    return lax.dot_general(a, b, (((1,), (0,)), ((), ())),
                           preferred_element_type=F32,
                           precision=lax.Precision.HIGHEST)


def _gather_rows(table, q):
    n, c = table.shape
    acc = jnp.zeros((q.shape[0], c), F32)
    for ch in range(n // 128):
        ioc = lax.broadcasted_iota(I32, (1, 128), 1) + ch * 128
        oh = (q == ioc).astype(F32)
        acc = acc + _matmul(oh, table[ch * 128:(ch + 1) * 128, :])
    return acc


def _gather_vals(vals2d, q):
    r = vals2d.shape[0]
    acc = jnp.zeros((q.shape[0], 1), F32)
    for ch in range(r):
        ioc = lax.broadcasted_iota(I32, (1, 128), 1) + ch * 128
        oh = (q == ioc).astype(F32)
        acc = acc + lax.dot_general(oh, vals2d[ch:ch + 1, :],
                                    (((1,), (1,)), ((), ())),
                                    preferred_element_type=F32,
                                    precision=lax.Precision.HIGHEST)
    return acc


def _compact_rows(table, prefix, mask, kp):
    n, c = table.shape
    jio = lax.broadcasted_iota(I32, (kp, 1), 0).astype(F32)
    acc = jnp.zeros((kp, c), F32)
    for ch in range(n // 128):
        pr = prefix[ch:ch + 1, :]
        mr = mask[ch:ch + 1, :]
        oh = jnp.where(mr, (jio == pr).astype(F32), jnp.zeros((kp, 128), F32))
        acc = acc + _matmul(oh, table[ch * 128:(ch + 1) * 128, :])
    return acc


def _bn_relu(h, g, b):
    return jnp.maximum((h * BN_INV) * g + b, 0.0)


def _matmul_conv(a, b):
    return lax.dot_general(a, b, (((1,), (0,)), ((), ())),
                           preferred_element_type=F32)


def _conv_tail(h, nr4, w, g, b, valid_n):
    n = h.shape[0]
    iot = lax.broadcasted_iota(I32, (n, 1), 0)
    cols = []
    for j in range(4):
        c = nr4[:, j:j + 1]
        cols.append(jnp.where(c < 0, iot, c))
    q = jnp.concatenate(cols, axis=0)
    gat = _gather_rows(h, q)
    g0, g1_, g2_, g3_ = (gat[j * n:(j + 1) * n, :] for j in range(4))
    feats = jnp.concatenate([
        h, jnp.minimum(g0, g1_), jnp.maximum(g0, g1_),
        jnp.minimum(g2_, g3_), jnp.maximum(g2_, g3_)], axis=1)
    return _bn_relu(_matmul_conv(feats, w), g, b)


def _pool_tail(h, nr4, n_valid, k, kp):
    n, c = h.shape
    r = n // 128
    s = jnp.sqrt(jnp.sum(h * h, axis=1))
    s2d = s.reshape(r, 128)
    flat = (lax.broadcasted_iota(I32, (r, 128), 0) * 128
            + lax.broadcasted_iota(I32, (r, 128), 1))
    valid = flat < n_valid
    s2d = jnp.where(valid, s2d, jnp.zeros((r, 128), F32))
    mask, prefix = _select_topk(s2d, valid, k)
    iot = lax.broadcasted_iota(I32, (n, 1), 0)
    nr4f = jnp.where(nr4 < 0, jnp.broadcast_to(iot, nr4.shape), nr4)
    table = jnp.concatenate([h, nr4f.astype(F32)], axis=1)
    comp = _compact_rows(table, prefix, mask, kp)
    hp = comp[:, :c]
    nbrk = comp[:, c:c + 4].astype(I32)
    remap = jnp.where(mask, prefix, jnp.full((r, 128), -1.0, F32))
    qs = jnp.concatenate([nbrk[:, j:j + 1] for j in range(4)], axis=0)
    qs = jnp.minimum(jnp.maximum(qs, 0), n - 1)
    nrg = _gather_vals(remap, qs)
    nr4p = jnp.concatenate(
        [nrg[j * kp:(j + 1) * kp, :] for j in range(4)], axis=1).astype(I32)
    return hp, nr4p


def _e_body(xk_ref, nk0_ref, nk1_ref, nk2_ref, nk3_ref, nr_ref,
            w1_ref, g1_ref, b1_ref,
            w2_ref, g2_ref, b2_ref, w3_ref, g3_ref, b3_ref,
            w4_ref, g4_ref, b4_ref, wh1_ref, gh_ref, bh_ref,
            wh2_ref, bh2_ref, out_ref):
    nr4 = nr_ref[...]
    h1 = _conv1_block(xk_ref[...], nk0_ref[...], nk1_ref[...], nk2_ref[...],
                      nk3_ref[...], w1_ref[...], g1_ref, b1_ref)
    h2 = _conv_tail(h1, nr4, w2_ref[...], g2_ref[0:1, :], b2_ref[0:1, :], K1)
    h2p, nr4b = _pool_tail(h2, nr4, K1, K2, K2P)
    h3 = _conv_tail(h2p, nr4b, w3_ref[...], g3_ref[0:1, :], b3_ref[0:1, :], K2)
    h3p, nr4c = _pool_tail(h3, nr4b, K2, K3, K3P)
    h4 = _conv_tail(h3p, nr4c, w4_ref[...], g4_ref[0:1, :], b4_ref[0:1, :], K3)
    vio = lax.broadcasted_iota(I32, (K3P, 1), 0) < K3
    hm = jnp.where(vio, h4, jnp.full(h4.shape, -1.0, F32))
    gmax = jnp.max(hm, axis=0, keepdims=True)
    z = _bn_relu(_matmul_conv(gmax, wh1_ref[...]), gh_ref[0:1, :],
                 bh_ref[0:1, :])
    logits = _matmul_conv(z, wh2_ref[...]) + bh2_ref[0:1, :]
    out_ref[...] = jnp.broadcast_to(logits, (8, 128))


def _tail_call(xk, nks, nrm, w1p, g1r, b1r, w2, g2r, b2r, w3, g3r, b3r,
               w4, g4r, b4r, wh1, ghr, bhr, wh2p, bh2r):
    return pl.pallas_call(
        _e_body,
        out_shape=jax.ShapeDtypeStruct((8, 128), F32),
    )(xk, *nks, nrm, w1p, g1r, b1r, w2, g2r, b2r, w3, g3r, b3r,
      w4, g4r, b4r, wh1, ghr, bhr, wh2p, bh2r)


def _row8(v, width):
    return jnp.broadcast_to(v.reshape(1, -1), (8, width))


def kernel(x, neighbor_idx, W1, g1, b1, W2, g2, b2, W3, g3, b3,
           W4, g4, b4, Wh1, gh, bh, Wh2, bh2):
    x16 = jnp.zeros((EP, 16), F32).at[:E, :5].set(x)
    nbc = [jnp.zeros((EP,), I32).at[:E].set(neighbor_idx[:, j])
           for j in range(4)]
    w1p = jnp.zeros((32, 64), F32).at[:25, :].set(W1)
    wh2p = jnp.zeros((256, 128), F32).at[:, :30].set(Wh2)
    g1r, b1r = _row8(g1, 64), _row8(b1, 64)
    g2r, b2r = _row8(g2, 128), _row8(b2, 128)
    g3r, b3r = _row8(g3, 256), _row8(b3, 256)
    g4r, b4r = _row8(g4, 512), _row8(b4, 512)
    ghr, bhr = _row8(gh, 256), _row8(bh, 256)
    bh2r = jnp.zeros((8, 128), F32).at[:, :30].set(
        jnp.broadcast_to(bh2.reshape(1, -1), (8, 30)))

    nbrf = _sc_gather_nbrs_fn()(x16, *nbc)
    scores = _scores_call(x16, nbrf, w1p, g1r, b1r)
    remap2d = _remap_call(scores)
    remap = remap2d.reshape(EP)
    keep = _sc_scatter_keep_fn()(remap)
    xk, nk0, nk1, nk2, nk3, nr0, nr1, nr2, nr3 = _sc_gather_kept_fn()(
        keep, x16, *nbrf, *nbc, remap)
    nrm = jnp.stack([nr0, nr1, nr2, nr3], axis=1)
    out = _tail_call(xk, (nk0, nk1, nk2, nk3), nrm, w1p, g1r, b1r,
                     W2, g2r, b2r,
                     W3, g3r, b3r, W4, g4r, b4r, Wh1, ghr, bhr, wh2p, bh2r)
    return out[:1, :30]


# speedup vs baseline: 158.6231x; 1.4655x over previous
import functools
import math

import jax
import jax.numpy as jnp
from jax import lax
from jax.experimental import pallas as pl
from jax.experimental.pallas import tpu as pltpu
from jax.experimental.pallas import tpu_sc as plsc

E = 100000
EP = 100352
NW = 32
RA = EP // NW
K1, K1P = 1500, 1536
K2, K2P = 750, 768
K3, K3P = 375, 384
KB = K1P // NW
KEEPN = 2048
BN_INV = 1.0 / math.sqrt(1.0 + 1e-5)
C_BLK = 2048
F32 = jnp.float32
I32 = jnp.int32


def _mesh():
    return plsc.VectorSubcoreMesh(core_axis_name="c", subcore_axis_name="s")


def _wid():
    return lax.axis_index("s") * 2 + lax.axis_index("c")


@functools.cache
def _sc_gather_nbrs_fn():
    @functools.partial(
        pl.kernel, mesh=_mesh(),
        compiler_params=pltpu.CompilerParams(use_tc_tiling_on_sc=False),
        out_type=[jax.ShapeDtypeStruct((EP, 16), F32)] * 4,
        scratch_types=[pltpu.VMEM((RA,), I32), pltpu.VMEM((RA, 16), F32),
                       pltpu.SemaphoreType.DMA],
    )
    def _sc_gather_nbrs(x16, i0, i1, i2, i3, o0, o1, o2, o3,
                        idx_v, rows_v, sem):
        base = _wid() * RA
        for ih, oh in ((i0, o0), (i1, o1), (i2, o2), (i3, o3)):
            pltpu.sync_copy(ih.at[pl.ds(base, RA)], idx_v)
            pltpu.async_copy(x16.at[idx_v], rows_v, sem).wait()
            pltpu.sync_copy(rows_v, oh.at[pl.ds(base, RA)])
    return _sc_gather_nbrs


def _conv1_block(xc, n0, n1, n2, n3, w, g, b):
    mn01 = jnp.minimum(n0[:, 0:5], n1[:, 0:5])
    mx01 = jnp.maximum(n0[:, 0:5], n1[:, 0:5])
    mn23 = jnp.minimum(n2[:, 0:5], n3[:, 0:5])
    mx23 = jnp.maximum(n2[:, 0:5], n3[:, 0:5])
    z = jnp.zeros((xc.shape[0], 7), F32)
    feats = jnp.concatenate([xc[:, 0:5], mn01, mx01, mn23, mx23, z], axis=1)
    h = lax.dot_general(feats, w, (((1,), (0,)), ((), ())),
                        preferred_element_type=F32)
    h = (h * BN_INV) * g[0:1, :] + b[0:1, :]
    return jnp.maximum(h, 0.0)


def _cd_body(x_ref, n0_ref, n1_ref, n2_ref, n3_ref, w_ref, g_ref, b_ref,
             out_ref, s_acc):
    i = pl.program_id(0)
    h = _conv1_block(x_ref[...], n0_ref[...], n1_ref[...], n2_ref[...],
                     n3_ref[...], w_ref[...], g_ref, b_ref)
    s = jnp.sqrt(jnp.sum(h * h, axis=1))
    rb = C_BLK // 128
    s_acc[pl.ds(i * rb, rb), :] = s.reshape(rb, 128)

    @pl.when(i == EP // C_BLK - 1)
    def _finish():
        sc = s_acc[...]
        flat = (lax.broadcasted_iota(I32, sc.shape, 0) * 128
                + lax.broadcasted_iota(I32, sc.shape, 1))
        mask, prefix = _select_topk(sc, flat < E, K1)
        out_ref[...] = jnp.where(mask, prefix.astype(I32),
                                 jnp.full(sc.shape, -1, I32))


def _scores_remap_call(x16, nbrf, w1p, g1r, b1r):
    blk16 = pl.BlockSpec((C_BLK, 16), lambda i: (i, 0))
    return pl.pallas_call(
        _cd_body,
        grid=(EP // C_BLK,),
        in_specs=[
            blk16, blk16, blk16, blk16, blk16,
            pl.BlockSpec((32, 64), lambda i: (0, 0)),
            pl.BlockSpec((8, 64), lambda i: (0, 0)),
            pl.BlockSpec((8, 64), lambda i: (0, 0)),
        ],
        out_specs=pl.BlockSpec((EP // 128, 128), lambda i: (0, 0)),
        out_shape=jax.ShapeDtypeStruct((EP // 128, 128), I32),
        scratch_shapes=[pltpu.VMEM((EP // 128, 128), F32)],
    )(x16, *nbrf, w1p, g1r, b1r)


def _excl_cumsum(a):
    r = a.shape[0]
    u = (lax.broadcasted_iota(I32, (128, 128), 0)
         < lax.broadcasted_iota(I32, (128, 128), 1)).astype(F32)
    within = lax.dot_general(a, u, (((1,), (0,)), ((), ())),
                             preferred_element_type=F32,
                             precision=lax.Precision.HIGHEST)
    rs = jnp.sum(a, axis=1, keepdims=True)
    slt = (lax.broadcasted_iota(I32, (r, r), 0)
           > lax.broadcasted_iota(I32, (r, r), 1)).astype(F32)
    rowex = lax.dot_general(slt, rs, (((1,), (0,)), ((), ())),
                            preferred_element_type=F32,
                            precision=lax.Precision.HIGHEST)
    return rowex + within


def _select_topk(scores2d, valid, k):
    ints = jnp.where(valid, lax.bitcast_convert_type(scores2d, I32),
                     jnp.full(scores2d.shape, -1, I32))

    def bs(_, c):
        lo, hi = c
        d = hi - lo
        mid = lo + (d >> 1) + (d & 1)
        cnt = jnp.sum((ints >= mid).astype(I32))
        ok = cnt >= k
        return (jnp.where(ok, mid, lo), jnp.where(ok, hi, mid - 1))

    lo, _ = lax.fori_loop(0, 31, bs, (jnp.int32(0), jnp.int32(2**31 - 1)))
    gt = ints > lo
    eq = ints == lo
    need = (k - jnp.sum(gt.astype(I32))).astype(F32)
    eqr = _excl_cumsum(eq.astype(F32))
    mask = gt | (eq & (eqr < need))
    prefix = _excl_cumsum(mask.astype(F32))
    return mask, prefix


RB = EP // 16


@functools.cache
def _sc_pool_fn():
  @functools.partial(
    pl.kernel, mesh=_mesh(),
    compiler_params=pltpu.CompilerParams(use_tc_tiling_on_sc=False),
    out_type=[jax.ShapeDtypeStruct((K1P, 16), F32)] * 5
             + [jax.ShapeDtypeStruct((K1P,), I32)] * 4,
    scratch_types=[pltpu.VMEM((RB,), I32), pltpu.VMEM((RB,), I32),
                   pltpu.VMEM((RB,), I32),
                   pltpu.VMEM_SHARED((KEEPN + EP,), I32),
                   pltpu.VMEM((KB,), I32), pltpu.VMEM((KB,), I32),
                   pltpu.VMEM((KB, 16), F32),
                   pltpu.VMEM((KB,), I32), pltpu.VMEM((KB,), I32),
                   pltpu.VMEM((KB,), I32), pltpu.SemaphoreType.DMA],
  )
  def _sc_pool(remap, x16, n0, n1, n2, n3, nb0, nb1, nb2, nb3,
               xk, nk0, nk1, nk2, nk3, nr0, nr1, nr2, nr3,
               rem_v, dest_v, val_v, keep_sh,
               kraw, kidx, rowbuf, cb, cidx, rb, sem):
    sid = lax.axis_index("s")
    wid = sid * 2 + lax.axis_index("c")
    sbase = sid * RB
    pltpu.sync_copy(remap.at[pl.ds(sbase, RB)], rem_v)

    def step(s, _):
        r = rem_v[pl.ds(s * 16, 16)]
        gi = sbase + s * 16 + lax.iota(I32, 16)
        dest_v[pl.ds(s * 16, 16)] = jnp.where(r >= 0, r, KEEPN + gi)
        val_v[pl.ds(s * 16, 16)] = gi
        return 0

    lax.fori_loop(0, RB // 16, step, 0)
    pltpu.sync_copy(val_v, keep_sh.at[dest_v])
    plsc.subcore_barrier()

    base = wid * KB
    pltpu.sync_copy(keep_sh.at[pl.ds(base, KB)], kraw)
    for s in range(KB // 16):
        v = kraw[pl.ds(s * 16, 16)]
        kidx[pl.ds(s * 16, 16)] = jnp.minimum(
            jnp.maximum(v, 0), jnp.full((16,), E - 1, I32))
    for tab, dst in ((x16, xk), (n0, nk0), (n1, nk1), (n2, nk2), (n3, nk3)):
        pltpu.async_copy(tab.at[kidx], rowbuf, sem).wait()
        pltpu.sync_copy(rowbuf, dst.at[pl.ds(base, KB)])
    for nb, nr in ((nb0, nr0), (nb1, nr1), (nb2, nr2), (nb3, nr3)):
        pltpu.async_copy(nb.at[kidx], cb, sem).wait()
        for s in range(KB // 16):
            v = cb[pl.ds(s * 16, 16)]
            cidx[pl.ds(s * 16, 16)] = jnp.minimum(
                jnp.maximum(v, 0), jnp.full((16,), E - 1, I32))
        pltpu.async_copy(remap.at[cidx], rb, sem).wait()
        pltpu.sync_copy(rb, nr.at[pl.ds(base, KB)])
  return _sc_pool


def _matmul(a, b):
    return lax.dot_general(a, b, (((1,), (0,)), ((), ())),
                           preferred_element_type=F32,
                           precision=lax.Precision.HIGHEST)


def _gather_rows(table, q):
    n, c = table.shape
    acc = jnp.zeros((q.shape[0], c), F32)
    for ch in range(n // 128):
        ioc = lax.broadcasted_iota(I32, (1, 128), 1) + ch * 128
        oh = (q == ioc).astype(F32)
        acc = acc + _matmul(oh, table[ch * 128:(ch + 1) * 128, :])
    return acc


def _gather_vals(vals2d, q):
    r = vals2d.shape[0]
    acc = jnp.zeros((q.shape[0], 1), F32)
    for ch in range(r):
        ioc = lax.broadcasted_iota(I32, (1, 128), 1) + ch * 128
        oh = (q == ioc).astype(F32)
        acc = acc + lax.dot_general(oh, vals2d[ch:ch + 1, :],
                                    (((1,), (1,)), ((), ())),
                                    preferred_element_type=F32,
                                    precision=lax.Precision.HIGHEST)
    return acc


def _compact_rows(table, prefix, mask, kp):
    n, c = table.shape
    jio = lax.broadcasted_iota(I32, (kp, 1), 0).astype(F32)
    acc = jnp.zeros((kp, c), F32)
    for ch in range(n // 128):
        pr = prefix[ch:ch + 1, :]
        mr = mask[ch:ch + 1, :]
        oh = jnp.where(mr, (jio == pr).astype(F32), jnp.zeros((kp, 128), F32))
        acc = acc + _matmul(oh, table[ch * 128:(ch + 1) * 128, :])
    return acc


def _bn_relu(h, g, b):
    return jnp.maximum((h * BN_INV) * g + b, 0.0)


def _matmul_conv(a, b):
    return lax.dot_general(a, b, (((1,), (0,)), ((), ())),
                           preferred_element_type=F32)


def _conv_tail(h, nr4, w, g, b, valid_n):
    n = h.shape[0]
    iot = lax.broadcasted_iota(I32, (n, 1), 0)
    cols = []
    for j in range(4):
        c = nr4[:, j:j + 1]
        cols.append(jnp.where(c < 0, iot, c))
    q = jnp.concatenate(cols, axis=0)
    gat = _gather_rows(h, q)
    g0, g1_, g2_, g3_ = (gat[j * n:(j + 1) * n, :] for j in range(4))
    feats = jnp.concatenate([
        h, jnp.minimum(g0, g1_), jnp.maximum(g0, g1_),
        jnp.minimum(g2_, g3_), jnp.maximum(g2_, g3_)], axis=1)
    return _bn_relu(_matmul_conv(feats, w), g, b)


def _pool_tail(h, nr4, n_valid, k, kp):
    n, c = h.shape
    r = n // 128
    s = jnp.sqrt(jnp.sum(h * h, axis=1))
    s2d = s.reshape(r, 128)
    flat = (lax.broadcasted_iota(I32, (r, 128), 0) * 128
            + lax.broadcasted_iota(I32, (r, 128), 1))
    valid = flat < n_valid
    s2d = jnp.where(valid, s2d, jnp.zeros((r, 128), F32))
    mask, prefix = _select_topk(s2d, valid, k)
    iot = lax.broadcasted_iota(I32, (n, 1), 0)
    nr4f = jnp.where(nr4 < 0, jnp.broadcast_to(iot, nr4.shape), nr4)
    table = jnp.concatenate([h, nr4f.astype(F32)], axis=1)
    comp = _compact_rows(table, prefix, mask, kp)
    hp = comp[:, :c]
    nbrk = comp[:, c:c + 4].astype(I32)
    remap = jnp.where(mask, prefix, jnp.full((r, 128), -1.0, F32))
    qs = jnp.concatenate([nbrk[:, j:j + 1] for j in range(4)], axis=0)
    qs = jnp.minimum(jnp.maximum(qs, 0), n - 1)
    nrg = _gather_vals(remap, qs)
    nr4p = jnp.concatenate(
        [nrg[j * kp:(j + 1) * kp, :] for j in range(4)], axis=1).astype(I32)
    return hp, nr4p


def _e_body(xk_ref, nk0_ref, nk1_ref, nk2_ref, nk3_ref, nr_ref,
            w1_ref, g1_ref, b1_ref,
            w2_ref, g2_ref, b2_ref, w3_ref, g3_ref, b3_ref,
            w4_ref, g4_ref, b4_ref, wh1_ref, gh_ref, bh_ref,
            wh2_ref, bh2_ref, out_ref):
    nr4 = nr_ref[...]
    h1 = _conv1_block(xk_ref[...], nk0_ref[...], nk1_ref[...], nk2_ref[...],
                      nk3_ref[...], w1_ref[...], g1_ref, b1_ref)
    h2 = _conv_tail(h1, nr4, w2_ref[...], g2_ref[0:1, :], b2_ref[0:1, :], K1)
    h2p, nr4b = _pool_tail(h2, nr4, K1, K2, K2P)
    h3 = _conv_tail(h2p, nr4b, w3_ref[...], g3_ref[0:1, :], b3_ref[0:1, :], K2)
    h3p, nr4c = _pool_tail(h3, nr4b, K2, K3, K3P)
    h4 = _conv_tail(h3p, nr4c, w4_ref[...], g4_ref[0:1, :], b4_ref[0:1, :], K3)
    vio = lax.broadcasted_iota(I32, (K3P, 1), 0) < K3
    hm = jnp.where(vio, h4, jnp.full(h4.shape, -1.0, F32))
    gmax = jnp.max(hm, axis=0, keepdims=True)
    z = _bn_relu(_matmul_conv(gmax, wh1_ref[...]), gh_ref[0:1, :],
                 bh_ref[0:1, :])
    logits = _matmul_conv(z, wh2_ref[...]) + bh2_ref[0:1, :]
    out_ref[...] = jnp.broadcast_to(logits, (8, 128))


def _tail_call(xk, nks, nrm, w1p, g1r, b1r, w2, g2r, b2r, w3, g3r, b3r,
               w4, g4r, b4r, wh1, ghr, bhr, wh2p, bh2r):
    return pl.pallas_call(
        _e_body,
        out_shape=jax.ShapeDtypeStruct((8, 128), F32),
    )(xk, *nks, nrm, w1p, g1r, b1r, w2, g2r, b2r, w3, g3r, b3r,
      w4, g4r, b4r, wh1, ghr, bhr, wh2p, bh2r)


def _row8(v, width):
    return jnp.broadcast_to(v.reshape(1, -1), (8, width))


def kernel(x, neighbor_idx, W1, g1, b1, W2, g2, b2, W3, g3, b3,
           W4, g4, b4, Wh1, gh, bh, Wh2, bh2):
    x16 = jnp.zeros((EP, 16), F32).at[:E, :5].set(x)
    nbc = [jnp.zeros((EP,), I32).at[:E].set(neighbor_idx[:, j])
           for j in range(4)]
    w1p = jnp.zeros((32, 64), F32).at[:25, :].set(W1)
    wh2p = jnp.zeros((256, 128), F32).at[:, :30].set(Wh2)
    g1r, b1r = _row8(g1, 64), _row8(b1, 64)
    g2r, b2r = _row8(g2, 128), _row8(b2, 128)
    g3r, b3r = _row8(g3, 256), _row8(b3, 256)
    g4r, b4r = _row8(g4, 512), _row8(b4, 512)
    ghr, bhr = _row8(gh, 256), _row8(bh, 256)
    bh2r = jnp.zeros((8, 128), F32).at[:, :30].set(
        jnp.broadcast_to(bh2.reshape(1, -1), (8, 30)))

    nbrf = _sc_gather_nbrs_fn()(x16, *nbc)
    remap2d = _scores_remap_call(x16, nbrf, w1p, g1r, b1r)
    remap = remap2d.reshape(EP)
    xk, nk0, nk1, nk2, nk3, nr0, nr1, nr2, nr3 = _sc_pool_fn()(
        remap, x16, *nbrf, *nbc)
    nrm = jnp.stack([nr0, nr1, nr2, nr3], axis=1)
    out = _tail_call(xk, (nk0, nk1, nk2, nk3), nrm, w1p, g1r, b1r,
                     W2, g2r, b2r,
                     W3, g3r, b3r, W4, g4r, b4r, Wh1, ghr, bhr, wh2p, bh2r)
    return out[:1, :30]
